# baseline (speedup 1.0000x reference)
import functools

import jax
import jax.numpy as jnp
from jax import lax
from jax.experimental import pallas as pl
from jax.experimental.pallas import tpu as pltpu
from jax.experimental.pallas import tpu_sc as plsc

N_NODES = 10000
N_EDGES = 160000
F_IN = 256
F_HID = 128

NCORE = 2
NSUB = 16
NW = NCORE * NSUB
CHUNK = 128
EPW = 5120
NCHUNK = EPW // CHUNK
E_PAD = EPW * NW
N_ACC = 10112
ROWS_ACC = N_ACC // NSUB
ROW_BLK = 1000
NBUF = 2
NC0 = 78
NC1 = 2


def _vmesh():
    return plsc.VectorSubcoreMesh(core_axis_name="c", subcore_axis_name="s")


def _sc_deg(dstp, ones128, zeros128):
    @functools.partial(
        pl.kernel,
        out_type=jax.ShapeDtypeStruct((NCORE, N_ACC, F_HID), jnp.float32),
        mesh=_vmesh(),
        scratch_types=[
            pltpu.VMEM((NCHUNK, CHUNK), jnp.int32),
            pltpu.VMEM((CHUNK, F_HID), jnp.float32),
            pltpu.VMEM_SHARED((N_ACC, F_HID), jnp.float32),
        ],
    )
    def k(dst_hbm, ones_hbm, zeros_hbm, deg_hbm, dst_v, ones_v, acc_sh):
        c = lax.axis_index("c")
        s = lax.axis_index("s")
        w = c * NSUB + s
        pltpu.sync_copy(dst_hbm.at[w], dst_v)
        pltpu.sync_copy(ones_hbm, ones_v)
        pltpu.sync_copy(zeros_hbm.at[pl.ds(s * ROWS_ACC, ROWS_ACC)],
                        acc_sh.at[pl.ds(s * ROWS_ACC, ROWS_ACC)])
        plsc.subcore_barrier()

        @pl.loop(0, NCHUNK)
        def _(j):
            pltpu.sync_copy(ones_v, acc_sh.at[dst_v.at[j]], add=True)

        plsc.subcore_barrier()
        pltpu.sync_copy(acc_sh.at[pl.ds(s * ROWS_ACC, ROWS_ACC)],
                        deg_hbm.at[c, pl.ds(s * ROWS_ACC, ROWS_ACC)])

    return k(dstp, ones128, zeros128)


def _sc_scatter(z, srcp, zeros128):
    @functools.partial(
        pl.kernel,
        out_type=jax.ShapeDtypeStruct((NCORE, N_ACC, F_HID), jnp.float32),
        mesh=_vmesh(),
        scratch_types=[
            pltpu.VMEM((NC0, CHUNK), jnp.int32),
            pltpu.VMEM((NBUF, CHUNK), jnp.int32),
            pltpu.VMEM((NBUF, CHUNK), jnp.int32),
            pltpu.VMEM((NBUF, CHUNK, F_HID), jnp.float32),
            pltpu.VMEM_SHARED((N_ACC, F_HID), jnp.float32),
            pltpu.SemaphoreType.DMA((NBUF,)),
        ],
    )
    def k(z_hbm, packed_hbm, zeros_hbm, out_hbm,
          packed_v, sidx, didx, buf, acc_sh, gsem):
        c = lax.axis_index("c")
        s = lax.axis_index("s")
        w = c * NSUB + s
        nc = lax.select(c == 0, NC0, NC1)

        def unpack(j, b):
            for g in range(CHUNK // 16):
                v = packed_v[j, pl.ds(g * 16, 16)]
                sidx[b, pl.ds(g * 16, 16)] = lax.bitwise_and(v, 0xFFFF)
                didx[b, pl.ds(g * 16, 16)] = lax.shift_right_logical(v, 16)

        pltpu.sync_copy(zeros_hbm.at[pl.ds(s * ROWS_ACC, ROWS_ACC)],
                        acc_sh.at[pl.ds(s * ROWS_ACC, ROWS_ACC)])
        pltpu.sync_copy(packed_hbm.at[w], packed_v)
        plsc.subcore_barrier()

        for b in range(NBUF):
            unpack(b, b)
            pltpu.async_copy(z_hbm.at[sidx.at[b]], buf.at[b], gsem.at[b])

        @pl.loop(0, nc, step=2)
        def _(j):
            for b in range(2):
                pltpu.make_async_copy(z_hbm.at[sidx.at[b]],
                                      buf.at[b], gsem.at[b]).wait()
                pltpu.sync_copy(buf.at[b], acc_sh.at[didx.at[b]], add=True)

                @pl.when(j + b + 2 < nc)
                def _():
                    unpack(j + b + 2, b)
                    pltpu.async_copy(z_hbm.at[sidx.at[b]],
                                     buf.at[b], gsem.at[b])

        plsc.subcore_barrier()
        pltpu.sync_copy(acc_sh.at[pl.ds(s * ROWS_ACC, ROWS_ACC)],
                        out_hbm.at[c, pl.ds(s * ROWS_ACC, ROWS_ACC)])

    return k(z, srcp, zeros128)


def _dinv_block(d_ref):
    d = d_ref[0][:, 0:1] + d_ref[1][:, 0:1] + 1.0
    return lax.rsqrt(d)


def _deg_spec():
    return pl.BlockSpec((NCORE, ROW_BLK, F_HID), lambda i: (0, i, 0))


def _tc_lin1(x, W1, deg):
    def body(x_ref, w_ref, d_ref, o_ref):
        y = jnp.dot(x_ref[...], w_ref[...], preferred_element_type=jnp.float32)
        o_ref[...] = y * _dinv_block(d_ref)

    return pl.pallas_call(
        body,
        grid=(N_NODES // ROW_BLK,),
        in_specs=[
            pl.BlockSpec((ROW_BLK, F_IN), lambda i: (i, 0)),
            pl.BlockSpec((F_IN, F_HID), lambda i: (0, 0)),
            _deg_spec(),
        ],
        out_specs=pl.BlockSpec((ROW_BLK, F_HID), lambda i: (i, 0)),
        out_shape=jax.ShapeDtypeStruct((N_NODES, F_HID), jnp.float32),
    )(x, W1, deg)


def _tc_lin2(acc, z1, deg, b1, W3):
    def body(a_ref, z_ref, d_ref, b_ref, w_ref, o_ref):
        dinv = _dinv_block(d_ref)
        h = (a_ref[0] + a_ref[1] + z_ref[...]) * dinv + b_ref[...]
        h = jnp.maximum(h, 0.0)
        y = jnp.dot(h, w_ref[...], preferred_element_type=jnp.float32)
        o_ref[...] = y * dinv

    return pl.pallas_call(
        body,
        grid=(N_NODES // ROW_BLK,),
        in_specs=[
            pl.BlockSpec((NCORE, ROW_BLK, F_HID), lambda i: (0, i, 0)),
            pl.BlockSpec((ROW_BLK, F_HID), lambda i: (i, 0)),
            _deg_spec(),
            pl.BlockSpec((1, F_HID), lambda i: (0, 0)),
            pl.BlockSpec((F_HID, F_HID), lambda i: (0, 0)),
        ],
        out_specs=pl.BlockSpec((ROW_BLK, F_HID), lambda i: (i, 0)),
        out_shape=jax.ShapeDtypeStruct((N_NODES, F_HID), jnp.float32),
    )(acc, z1, deg, b1, W3)


def _tc_final(acc, z2, deg, b3):
    def body(a_ref, z_ref, d_ref, b_ref, o_ref):
        dinv = _dinv_block(d_ref)
        o_ref[...] = (a_ref[0] + a_ref[1] + z_ref[...]) * dinv + b_ref[...]

    return pl.pallas_call(
        body,
        grid=(N_NODES // ROW_BLK,),
        in_specs=[
            pl.BlockSpec((NCORE, ROW_BLK, F_HID), lambda i: (0, i, 0)),
            pl.BlockSpec((ROW_BLK, F_HID), lambda i: (i, 0)),
            _deg_spec(),
            pl.BlockSpec((1, F_HID), lambda i: (0, 0)),
        ],
        out_specs=pl.BlockSpec((ROW_BLK, F_HID), lambda i: (i, 0)),
        out_shape=jax.ShapeDtypeStruct((N_NODES, F_HID), jnp.float32),
    )(acc, z2, deg, b3)


def kernel(x, edge_index, W1, b1, W3, b3):
    src = edge_index[0]
    dst = edge_index[1]
    pad_src = jnp.zeros((E_PAD - N_EDGES,), jnp.int32)
    pad_dst = jnp.full((E_PAD - N_EDGES,), N_NODES, jnp.int32)
    dstp = jnp.concatenate([dst, pad_dst]).reshape(NW, NCHUNK, CHUNK)

    packed = jnp.bitwise_or(src, jnp.left_shift(dst, 16))
    pad_word = jnp.int32(N_NODES << 16)
    e0 = NSUB * NC0 * CHUNK
    e1cap = NSUB * NC1 * CHUNK
    a = packed[:e0].reshape(NSUB, NC0, CHUNK)
    b = jnp.concatenate(
        [packed[e0:], jnp.full((e0 + e1cap - N_EDGES,), pad_word, jnp.int32)]
    ).reshape(NSUB, NC1, CHUNK)
    b = jnp.concatenate(
        [b, jnp.full((NSUB, NC0 - NC1, CHUNK), pad_word, jnp.int32)], axis=1)
    srcp = jnp.concatenate([a, b], axis=0)
    ones128 = jnp.ones((CHUNK, F_HID), jnp.float32)
    zeros128 = jnp.zeros((N_ACC, F_HID), jnp.float32)

    deg = _sc_deg(dstp, ones128, zeros128)
    z1 = _tc_lin1(x, W1, deg)
    acc1 = _sc_scatter(z1, srcp, zeros128)
    z2 = _tc_lin2(acc1, z1, deg, b1.reshape(1, F_HID), W3)
    acc2 = _sc_scatter(z2, srcp, zeros128)
    return _tc_final(acc2, z2, deg, b3.reshape(1, F_HID))


# speedup vs baseline: 1.0019x; 1.0019x over previous
import functools

import jax
import jax.numpy as jnp
from jax import lax
from jax.experimental import pallas as pl
from jax.experimental.pallas import tpu as pltpu
from jax.experimental.pallas import tpu_sc as plsc

N_NODES = 10000
N_EDGES = 160000
F_IN = 256
F_HID = 128

NCORE = 2
NSUB = 16
NW = NCORE * NSUB
CHUNK = 128
EPW = 5120
NCHUNK = EPW // CHUNK
E_PAD = EPW * NW
N_ACC = 10112
ROWS_ACC = N_ACC // NSUB
ROW_BLK = 1000
NBUF = 2
NC0 = 72
NC1 = 8


def _vmesh():
    return plsc.VectorSubcoreMesh(core_axis_name="c", subcore_axis_name="s")


def _sc_deg(dstp, ones128, zeros128):
    @functools.partial(
        pl.kernel,
        out_type=jax.ShapeDtypeStruct((NCORE, N_ACC, F_HID), jnp.float32),
        mesh=_vmesh(),
        scratch_types=[
            pltpu.VMEM((NCHUNK, CHUNK), jnp.int32),
            pltpu.VMEM((CHUNK, F_HID), jnp.float32),
            pltpu.VMEM_SHARED((N_ACC, F_HID), jnp.float32),
        ],
    )
    def k(dst_hbm, ones_hbm, zeros_hbm, deg_hbm, dst_v, ones_v, acc_sh):
        c = lax.axis_index("c")
        s = lax.axis_index("s")
        w = c * NSUB + s
        pltpu.sync_copy(dst_hbm.at[w], dst_v)
        pltpu.sync_copy(ones_hbm, ones_v)
        pltpu.sync_copy(zeros_hbm.at[pl.ds(s * ROWS_ACC, ROWS_ACC)],
                        acc_sh.at[pl.ds(s * ROWS_ACC, ROWS_ACC)])
        plsc.subcore_barrier()

        @pl.loop(0, NCHUNK)
        def _(j):
            pltpu.sync_copy(ones_v, acc_sh.at[dst_v.at[j]], add=True)

        plsc.subcore_barrier()
        pltpu.sync_copy(acc_sh.at[pl.ds(s * ROWS_ACC, ROWS_ACC)],
                        deg_hbm.at[c, pl.ds(s * ROWS_ACC, ROWS_ACC)])

    return k(dstp, ones128, zeros128)


def _sc_scatter(z, srcp, zeros128):
    @functools.partial(
        pl.kernel,
        out_type=jax.ShapeDtypeStruct((NCORE, N_ACC, F_HID), jnp.float32),
        mesh=_vmesh(),
        scratch_types=[
            pltpu.VMEM((NC0, CHUNK), jnp.int32),
            pltpu.VMEM((NBUF, CHUNK), jnp.int32),
            pltpu.VMEM((NBUF, CHUNK), jnp.int32),
            pltpu.VMEM((NBUF, CHUNK, F_HID), jnp.float32),
            pltpu.VMEM_SHARED((N_ACC, F_HID), jnp.float32),
            pltpu.SemaphoreType.DMA((NBUF,)),
        ],
    )
    def k(z_hbm, packed_hbm, zeros_hbm, out_hbm,
          packed_v, sidx, didx, buf, acc_sh, gsem):
        c = lax.axis_index("c")
        s = lax.axis_index("s")
        w = c * NSUB + s
        nc = lax.select(c == 0, NC0, NC1)

        def unpack(j, b):
            for g in range(CHUNK // 16):
                v = packed_v[j, pl.ds(g * 16, 16)]
                sidx[b, pl.ds(g * 16, 16)] = lax.bitwise_and(v, 0xFFFF)
                didx[b, pl.ds(g * 16, 16)] = lax.shift_right_logical(v, 16)

        pltpu.sync_copy(zeros_hbm.at[pl.ds(s * ROWS_ACC, ROWS_ACC)],
                        acc_sh.at[pl.ds(s * ROWS_ACC, ROWS_ACC)])
        pltpu.sync_copy(packed_hbm.at[w], packed_v)
        plsc.subcore_barrier()

        for b in range(NBUF):
            unpack(b, b)
            pltpu.async_copy(z_hbm.at[sidx.at[b]], buf.at[b], gsem.at[b])

        @pl.loop(0, nc, step=2)
        def _(j):
            for b in range(2):
                pltpu.make_async_copy(z_hbm.at[sidx.at[b]],
                                      buf.at[b], gsem.at[b]).wait()
                pltpu.sync_copy(buf.at[b], acc_sh.at[didx.at[b]], add=True)

                @pl.when(j + b + 2 < nc)
                def _():
                    unpack(j + b + 2, b)
                    pltpu.async_copy(z_hbm.at[sidx.at[b]],
                                     buf.at[b], gsem.at[b])

        plsc.subcore_barrier()
        pltpu.sync_copy(acc_sh.at[pl.ds(s * ROWS_ACC, ROWS_ACC)],
                        out_hbm.at[c, pl.ds(s * ROWS_ACC, ROWS_ACC)])

    return k(z, srcp, zeros128)


def _dinv_block(d_ref):
    d = d_ref[0][:, 0:1] + d_ref[1][:, 0:1] + 1.0
    return lax.rsqrt(d)


def _deg_spec():
    return pl.BlockSpec((NCORE, ROW_BLK, F_HID), lambda i: (0, i, 0))


def _tc_lin1(x, W1, deg):
    def body(x_ref, w_ref, d_ref, o_ref):
        y = jnp.dot(x_ref[...], w_ref[...], preferred_element_type=jnp.float32)
        o_ref[...] = y * _dinv_block(d_ref)

    return pl.pallas_call(
        body,
        grid=(N_NODES // ROW_BLK,),
        in_specs=[
            pl.BlockSpec((ROW_BLK, F_IN), lambda i: (i, 0)),
            pl.BlockSpec((F_IN, F_HID), lambda i: (0, 0)),
            _deg_spec(),
        ],
        out_specs=pl.BlockSpec((ROW_BLK, F_HID), lambda i: (i, 0)),
        out_shape=jax.ShapeDtypeStruct((N_NODES, F_HID), jnp.float32),
    )(x, W1, deg)


def _tc_lin2(acc, z1, deg, b1, W3):
    def body(a_ref, z_ref, d_ref, b_ref, w_ref, o_ref):
        dinv = _dinv_block(d_ref)
        h = (a_ref[0] + a_ref[1] + z_ref[...]) * dinv + b_ref[...]
        h = jnp.maximum(h, 0.0)
        y = jnp.dot(h, w_ref[...], preferred_element_type=jnp.float32)
        o_ref[...] = y * dinv

    return pl.pallas_call(
        body,
        grid=(N_NODES // ROW_BLK,),
        in_specs=[
            pl.BlockSpec((NCORE, ROW_BLK, F_HID), lambda i: (0, i, 0)),
            pl.BlockSpec((ROW_BLK, F_HID), lambda i: (i, 0)),
            _deg_spec(),
            pl.BlockSpec((1, F_HID), lambda i: (0, 0)),
            pl.BlockSpec((F_HID, F_HID), lambda i: (0, 0)),
        ],
        out_specs=pl.BlockSpec((ROW_BLK, F_HID), lambda i: (i, 0)),
        out_shape=jax.ShapeDtypeStruct((N_NODES, F_HID), jnp.float32),
    )(acc, z1, deg, b1, W3)


def _tc_final(acc, z2, deg, b3):
    def body(a_ref, z_ref, d_ref, b_ref, o_ref):
        dinv = _dinv_block(d_ref)
        o_ref[...] = (a_ref[0] + a_ref[1] + z_ref[...]) * dinv + b_ref[...]

    return pl.pallas_call(
        body,
        grid=(N_NODES // ROW_BLK,),
        in_specs=[
            pl.BlockSpec((NCORE, ROW_BLK, F_HID), lambda i: (0, i, 0)),
            pl.BlockSpec((ROW_BLK, F_HID), lambda i: (i, 0)),
            _deg_spec(),
            pl.BlockSpec((1, F_HID), lambda i: (0, 0)),
        ],
        out_specs=pl.BlockSpec((ROW_BLK, F_HID), lambda i: (i, 0)),
        out_shape=jax.ShapeDtypeStruct((N_NODES, F_HID), jnp.float32),
    )(acc, z2, deg, b3)


def kernel(x, edge_index, W1, b1, W3, b3):
    src = edge_index[0]
    dst = edge_index[1]
    pad_dst = jnp.full((E_PAD - N_EDGES,), N_NODES, jnp.int32)
    dstp = jnp.concatenate([dst, pad_dst]).reshape(NW, NCHUNK, CHUNK)

    packed = jnp.bitwise_or(src, jnp.left_shift(dst, 16))
    pad_word = jnp.int32(N_NODES << 16)
    e0 = NSUB * NC0 * CHUNK
    e1cap = NSUB * NC1 * CHUNK
    a = packed[:e0].reshape(NSUB, NC0, CHUNK)
    b = jnp.concatenate(
        [packed[e0:], jnp.full((e0 + e1cap - N_EDGES,), pad_word, jnp.int32)]
    ).reshape(NSUB, NC1, CHUNK)
    b = jnp.concatenate(
        [b, jnp.full((NSUB, NC0 - NC1, CHUNK), pad_word, jnp.int32)], axis=1)
    srcp = jnp.concatenate([a, b], axis=0)
    ones128 = jnp.ones((CHUNK, F_HID), jnp.float32)
    zeros128 = jnp.zeros((N_ACC, F_HID), jnp.float32)

    deg = _sc_deg(dstp, ones128, zeros128)
    z1 = _tc_lin1(x, W1, deg)
    acc1 = _sc_scatter(z1, srcp, zeros128)
    z2 = _tc_lin2(acc1, z1, deg, b1.reshape(1, F_HID), W3)
    acc2 = _sc_scatter(z2, srcp, zeros128)
    return _tc_final(acc2, z2, deg, b3.reshape(1, F_HID))
